# baseline (speedup 1.0000x reference)
import functools

import jax
import jax.numpy as jnp
from jax import lax
from jax.experimental import pallas as pl
from jax.experimental.pallas import tpu as pltpu
from jax.experimental.pallas import tpu_sc as plsc

N = 10000
D = 128
E = 160000
R = 2
L = 16
NS = 16
N_PAD = 10240
ROWS_PER_SUB = N_PAD // NS
E_PER_SUB = E // NS
CHUNK = 80
NCHUNK = E_PER_SUB // CHUNK
NMAIN = ((NCHUNK - 2) // 3) * 3


def _sc_segment_sum(x, src0, dst0, src1, dst1):
    mesh = plsc.VectorSubcoreMesh(core_axis_name="c", subcore_axis_name="s")

    @functools.partial(
        pl.kernel,
        out_type=(jax.ShapeDtypeStruct((R, N_PAD, D), jnp.float32),
                  jax.ShapeDtypeStruct((R, N_PAD), jnp.float32)),
        mesh=mesh,
        scratch_types=[
            pltpu.VMEM((E_PER_SUB,), jnp.int32),
            pltpu.VMEM((CHUNK,), jnp.int32),
            pltpu.VMEM((CHUNK,), jnp.int32),
            pltpu.VMEM((CHUNK,), jnp.int32),
            pltpu.VMEM((CHUNK, D), jnp.float32),
            pltpu.VMEM((CHUNK, D), jnp.float32),
            pltpu.VMEM((CHUNK, D), jnp.float32),
            pltpu.VMEM((CHUNK,), jnp.float32),
            pltpu.VMEM((CHUNK,), jnp.float32),
            pltpu.VMEM_SHARED((N_PAD, D), jnp.float32),
            pltpu.VMEM_SHARED((N_PAD,), jnp.float32),
            pltpu.SemaphoreType.DMA,
            pltpu.SemaphoreType.DMA,
            pltpu.SemaphoreType.DMA,
            pltpu.SemaphoreType.DMA,
            pltpu.SemaphoreType.DMA,
            pltpu.SemaphoreType.DMA,
        ],
    )
    def sc_kernel(x_hbm, src0_hbm, dst0_hbm, src1_hbm, dst1_hbm,
                  out_hbm, deg_hbm,
                  src_v, dstb0, dstb1, dstb2, rows0, rows1, rows2,
                  ones_v, zeros_v, acc_sh, deg_sh,
                  sg0, sg1, sg2, sd0, sd1, sd2):
        c = lax.axis_index("c")
        s = lax.axis_index("s")
        ebase = s * E_PER_SUB

        one = jnp.full((L,), 1.0, jnp.float32)
        zero = jnp.zeros((L,), jnp.float32)

        @pl.loop(0, CHUNK, step=L)
        def _(i):
            ones_v[pl.ds(i, L)] = one
            zeros_v[pl.ds(i, L)] = zero

        @pl.loop(0, 64)
        def _(i):
            @pl.loop(0, D, step=L)
            def _(j):
                rows0[i, pl.ds(j, L)] = zero

        row0 = s * ROWS_PER_SUB

        @pl.loop(0, ROWS_PER_SUB, step=64)
        def _(r0):
            pltpu.sync_copy(rows0.at[pl.ds(0, 64)],
                            acc_sh.at[pl.ds(row0 + r0, 64)])
            pltpu.sync_copy(zeros_v.at[pl.ds(0, 64)],
                            deg_sh.at[pl.ds(row0 + r0, 64)])

        plsc.subcore_barrier()

        def run_relation(src_hbm, dst_hbm):
            pltpu.async_copy(
                src_hbm.at[pl.ds(ebase, E_PER_SUB)], src_v, sg0).wait()

            def gather(k, buf, sem):
                return pltpu.make_async_copy(
                    x_hbm.at[src_v.at[pl.ds(k * CHUNK, CHUNK)]], buf, sem)

            def dstcp(k, buf, sem):
                return pltpu.make_async_copy(
                    dst_hbm.at[pl.ds(ebase + k * CHUNK, CHUNK)], buf, sem)

            def consume(k, buf, dbuf, sg, sd):
                gather(k, buf, sg).wait()
                dstcp(k, dbuf, sd).wait()
                pltpu.sync_copy(buf, acc_sh.at[dbuf], add=True)
                pltpu.sync_copy(ones_v, deg_sh.at[dbuf], add=True)

            def prefetch(k, buf, dbuf, sg, sd):
                @pl.when(k < NCHUNK)
                def _():
                    dstcp(k, dbuf, sd).start()
                    gather(k, buf, sg).start()

            dstcp(0, dstb0, sd0).start()
            gather(0, rows0, sg0).start()
            dstcp(1, dstb1, sd1).start()
            gather(1, rows1, sg1).start()

            @pl.loop(0, NMAIN, step=3)
            def _(a):
                prefetch(a + 2, rows2, dstb2, sg2, sd2)
                consume(a, rows0, dstb0, sg0, sd0)
                prefetch(a + 3, rows0, dstb0, sg0, sd0)
                consume(a + 1, rows1, dstb1, sg1, sd1)
                prefetch(a + 4, rows1, dstb1, sg1, sd1)
                consume(a + 2, rows2, dstb2, sg2, sd2)

            consume(NMAIN, rows0, dstb0, sg0, sd0)
            consume(NMAIN + 1, rows1, dstb1, sg1, sd1)

        @pl.when(c == 0)
        def _():
            run_relation(src0_hbm, dst0_hbm)

        @pl.when(c == 1)
        def _():
            run_relation(src1_hbm, dst1_hbm)

        plsc.subcore_barrier()

        pltpu.sync_copy(acc_sh.at[pl.ds(row0, ROWS_PER_SUB)],
                        out_hbm.at[c, pl.ds(row0, ROWS_PER_SUB)])
        pltpu.sync_copy(deg_sh.at[pl.ds(row0, ROWS_PER_SUB)],
                        deg_hbm.at[c, pl.ds(row0, ROWS_PER_SUB)])

    return sc_kernel(x, src0, dst0, src1, dst1)


_BR = 1000
_DN = (((1,), (0,)), ((), ()))
_HP = lax.Precision.HIGHEST
_EB = 16384


def _detile_body(e0_ref, e1_ref, s0_ref, d0_ref, s1_ref, d1_ref):
    s0_ref[...] = e0_ref[0]
    d0_ref[...] = e0_ref[1]
    s1_ref[...] = e1_ref[0]
    d1_ref[...] = e1_ref[1]


def _detile(ei0, ei1):
    out1d = jax.ShapeDtypeStruct((E,), jnp.int32)
    return pl.pallas_call(
        _detile_body,
        grid=((E + _EB - 1) // _EB,),
        in_specs=[
            pl.BlockSpec((2, _EB), lambda i: (0, i)),
            pl.BlockSpec((2, _EB), lambda i: (0, i)),
        ],
        out_specs=[pl.BlockSpec((_EB,), lambda i: (i,))] * 4,
        out_shape=[out1d] * 4,
    )(ei0, ei1)


def _tc_pre_body(x_ref, lw_ref, b_ref, o_ref):
    o_ref[...] = lax.dot_general(
        x_ref[...], lw_ref[...], _DN, precision=_HP) + b_ref[...]


def _tc_pre(x, loop_w, h_bias_row):
    return pl.pallas_call(
        _tc_pre_body,
        grid=(N // _BR,),
        in_specs=[
            pl.BlockSpec((_BR, D), lambda i: (i, 0)),
            pl.BlockSpec((D, D), lambda i: (0, 0)),
            pl.BlockSpec((1, D), lambda i: (0, 0)),
        ],
        out_specs=pl.BlockSpec((_BR, D), lambda i: (i, 0)),
        out_shape=jax.ShapeDtypeStruct((N, D), jnp.float32),
    )(x, loop_w, h_bias_row)


def _tc_post_body(s_hbm, deg_ref, p_ref, w0_ref, w1_ref, o_ref,
                  s0_v, s1_v, sem0, sem1):
    i = pl.program_id(0)
    cp0 = pltpu.make_async_copy(
        s_hbm.at[0, pl.ds(i * _BR, _BR)], s0_v, sem0)
    cp1 = pltpu.make_async_copy(
        s_hbm.at[1, pl.ds(i * _BR, _BR)], s1_v, sem1)
    cp0.start()
    cp1.start()
    d0 = jnp.maximum(deg_ref[0], 1.0)
    d1 = jnp.maximum(deg_ref[1], 1.0)
    cp0.wait()
    acc = lax.dot_general(s0_v[...], w0_ref[...], _DN, precision=_HP) / d0
    cp1.wait()
    acc = acc + lax.dot_general(s1_v[...], w1_ref[...], _DN, precision=_HP) / d1
    o_ref[...] = jnp.tanh(acc + p_ref[...])


def _tc_post(S, deg3, P, W0, W1):
    return pl.pallas_call(
        _tc_post_body,
        grid=(N // _BR,),
        in_specs=[
            pl.BlockSpec(memory_space=pl.ANY),
            pl.BlockSpec((R, _BR, 1), lambda i: (0, i, 0)),
            pl.BlockSpec((_BR, D), lambda i: (i, 0)),
            pl.BlockSpec((D, D), lambda i: (0, 0)),
            pl.BlockSpec((D, D), lambda i: (0, 0)),
        ],
        out_specs=pl.BlockSpec((_BR, D), lambda i: (i, 0)),
        out_shape=jax.ShapeDtypeStruct((N, D), jnp.float32),
        scratch_shapes=[
            pltpu.VMEM((_BR, D), jnp.float32),
            pltpu.VMEM((_BR, D), jnp.float32),
            pltpu.SemaphoreType.DMA,
            pltpu.SemaphoreType.DMA,
        ],
    )(S, deg3, P, W0, W1)


def kernel(x, W, loop_weight, h_bias, edge_index_rel0, edge_index_rel1):
    src0, dst0, src1, dst1 = _detile(edge_index_rel0, edge_index_rel1)
    P = _tc_pre(x, loop_weight, h_bias.reshape(1, D))
    S, deg = _sc_segment_sum(x, src0, dst0, src1, dst1)
    deg3 = deg.reshape(R, N_PAD, 1)
    return _tc_post(S, deg3, P, W[0], W[1])
